# Initial kernel scaffold; baseline (speedup 1.0000x reference)
import jax
import jax.numpy as jnp
from jax.experimental import pallas as pl


def kernel(words, pos_contexts, neg_contexts, w_embedding, c_embedding):
    raise NotImplementedError("write your pallas kernel here")


# speedup vs baseline: 1.7576x; 1.7576x over previous
import functools

import jax
import jax.numpy as jnp
from jax import lax
from jax.experimental import pallas as pl
from jax.experimental.pallas import tpu as pltpu
from jax.experimental.pallas import tpu_sc as plsc

B = 16384
P = 20
N = 50
DIM = 64
CPAD = 80
E = 8
NW = 32
EPW = B // NW
NCHUNK = EPW // E
ROWS = E * CPAD
IDXW = 128
IDXROWS = ROWS // IDXW


def _sc_scores(words, ctx, w_embedding, c_embedding):
    mesh = plsc.VectorSubcoreMesh(core_axis_name="c", subcore_axis_name="s")

    @functools.partial(
        pl.kernel,
        out_type=jax.ShapeDtypeStruct((B, CPAD), jnp.float32),
        mesh=mesh,
        compiler_params=pltpu.CompilerParams(needs_layout_passes=False,
                                             use_tc_tiling_on_sc=False),
        scratch_types=[
            pltpu.VMEM((ROWS,), jnp.int32),
            pltpu.VMEM((ROWS,), jnp.int32),
            pltpu.VMEM((ROWS, DIM), jnp.float32),
            pltpu.VMEM((ROWS, DIM), jnp.float32),
            pltpu.VMEM((E,), jnp.int32),
            pltpu.VMEM((E,), jnp.int32),
            pltpu.VMEM((E, DIM), jnp.float32),
            pltpu.VMEM((E, DIM), jnp.float32),
            pltpu.VMEM((E, CPAD), jnp.float32),
            pltpu.SemaphoreType.DMA,
            pltpu.SemaphoreType.DMA,
        ],
    )
    def sc_kernel(words_hbm, ctx_hbm, wtab_hbm, ctab_hbm, out_hbm,
                  idx0, idx1, rows0, rows1, widx0, widx1, wrows0, wrows1,
                  scores_v, sem0, sem1):
        idx_v = (idx0, idx1)
        rows_v = (rows0, rows1)
        widx_v = (widx0, widx1)
        wrows_v = (wrows0, wrows1)
        sems = (sem0, sem1)
        wid = lax.axis_index("s") * 2 + lax.axis_index("c")
        base_e0 = wid * EPW

        def start_fetch(c, slot):
            base_e = base_e0 + c * E
            pltpu.sync_copy(ctx_hbm.at[pl.ds(base_e * CPAD, ROWS)],
                            idx_v[slot])
            pltpu.sync_copy(words_hbm.at[pl.ds(base_e, E)], widx_v[slot])
            for i in range(IDXROWS):
                pltpu.async_copy(
                    ctab_hbm.at[idx_v[slot].at[pl.ds(i * IDXW, IDXW)]],
                    rows_v[slot].at[pl.ds(i * IDXW, IDXW)],
                    sems[slot],
                )
            pltpu.async_copy(wtab_hbm.at[widx_v[slot]],
                             wrows_v[slot], sems[slot])

        def wait_fetch(slot):
            pltpu.make_async_copy(
                ctab_hbm.at[pl.ds(0, ROWS)], rows_v[slot], sems[slot]
            ).wait()
            pltpu.make_async_copy(
                wtab_hbm.at[pl.ds(0, E)], wrows_v[slot], sems[slot]
            ).wait()

        def compute(c, slot):
            base_e = base_e0 + c * E
            lanes = lax.iota(jnp.int32, 16)

            def elem_body(e, _):
                accs = [jnp.zeros((16,), jnp.float32)
                        for _ in range(CPAD // 16)]
                row0 = e * CPAD + lanes
                for k in range(DIM // 16):
                    wchunk = wrows_v[slot][e, pl.ds(k * 16, 16)]
                    for i in range(16):
                        d = k * 16 + i
                        wd = wchunk[i]
                        col_idx = jnp.full((16,), d, jnp.int32)
                        for g in range(CPAD // 16):
                            col = plsc.load_gather(
                                rows_v[slot], [row0 + g * 16, col_idx])
                            accs[g] = accs[g] + col * wd
                for g in range(CPAD // 16):
                    scores_v[e, pl.ds(g * 16, 16)] = accs[g]
                return 0

            lax.fori_loop(0, E, elem_body, 0)
            pltpu.sync_copy(scores_v, out_hbm.at[pl.ds(base_e, E)])

        start_fetch(0, 0)
        start_fetch(1, 1)

        def chunk_body(g, _):
            for b in range(2):
                c = g * 2 + b
                wait_fetch(b)
                compute(c, b)

                @pl.when(c + 2 < NCHUNK)
                def _():
                    start_fetch(c + 2, b)
            return 0

        lax.fori_loop(0, NCHUNK // 2, chunk_body, 0, unroll=False)

    return sc_kernel(words, ctx, w_embedding, c_embedding)


def _tc_loss(scores):
    blk = 2048

    def tc_body(s_ref, o_ref):
        s = s_ref[...]
        j = lax.broadcasted_iota(jnp.int32, s.shape, 1)
        pos = jnp.where(j < P, jax.nn.log_sigmoid(s), 0.0).sum(axis=1) / P
        neg = jnp.where((j >= P) & (j < P + N),
                        jax.nn.log_sigmoid(-s), 0.0).sum(axis=1) / N
        o_ref[...] = -(pos + neg)

    return pl.pallas_call(
        tc_body,
        grid=(B // blk,),
        in_specs=[pl.BlockSpec((blk, CPAD), lambda i: (i, 0))],
        out_specs=pl.BlockSpec((blk,), lambda i: (i,)),
        out_shape=jax.ShapeDtypeStruct((B,), jnp.float32),
    )(scores)


def kernel(words, pos_contexts, neg_contexts, w_embedding, c_embedding):
    pad = jnp.zeros((B, CPAD - P - N), jnp.int32)
    ctx = jnp.concatenate([pos_contexts, neg_contexts, pad],
                          axis=1).reshape(-1)
    scores = _sc_scores(words, ctx, w_embedding, c_embedding)
    return _tc_loss(scores)


# speedup vs baseline: 1.7703x; 1.0073x over previous
import functools

import jax
import jax.numpy as jnp
from jax import lax
from jax.experimental import pallas as pl
from jax.experimental.pallas import tpu as pltpu
from jax.experimental.pallas import tpu_sc as plsc

B = 16384
P = 20
N = 50
DIM = 64
CPAD = 80
E = 8
NW = 32
EPW = B // NW
NCHUNK = EPW // E
ROWS = E * CPAD
IDXW = 128
IDXROWS = ROWS // IDXW


def _sc_scores(words, ctx, w_embedding, c_embedding):
    mesh = plsc.VectorSubcoreMesh(core_axis_name="c", subcore_axis_name="s")

    @functools.partial(
        pl.kernel,
        out_type=jax.ShapeDtypeStruct((B, CPAD), jnp.float32),
        mesh=mesh,
        compiler_params=pltpu.CompilerParams(needs_layout_passes=False,
                                             use_tc_tiling_on_sc=False),
        scratch_types=[
            pltpu.VMEM((ROWS,), jnp.int32),
            pltpu.VMEM((ROWS,), jnp.int32),
            pltpu.VMEM((ROWS, DIM), jnp.float32),
            pltpu.VMEM((ROWS, DIM), jnp.float32),
            pltpu.VMEM((E,), jnp.int32),
            pltpu.VMEM((E,), jnp.int32),
            pltpu.VMEM((E, DIM), jnp.float32),
            pltpu.VMEM((E, DIM), jnp.float32),
            pltpu.VMEM((E, CPAD), jnp.float32),
            pltpu.SemaphoreType.DMA,
            pltpu.SemaphoreType.DMA,
        ],
    )
    def sc_kernel(words_hbm, ctx_hbm, wtab_hbm, ctab_hbm, out_hbm,
                  idx0, idx1, rows0, rows1, widx0, widx1, wrows0, wrows1,
                  scores_v, sem0, sem1):
        idx_v = (idx0, idx1)
        rows_v = (rows0, rows1)
        widx_v = (widx0, widx1)
        wrows_v = (wrows0, wrows1)
        sems = (sem0, sem1)
        wid = lax.axis_index("s") * 2 + lax.axis_index("c")
        base_e0 = wid * EPW

        def start_fetch(c, slot):
            base_e = base_e0 + c * E
            pltpu.sync_copy(ctx_hbm.at[pl.ds(base_e * CPAD, ROWS)],
                            idx_v[slot])
            pltpu.sync_copy(words_hbm.at[pl.ds(base_e, E)], widx_v[slot])
            for i in range(IDXROWS):
                pltpu.async_copy(
                    ctab_hbm.at[idx_v[slot].at[pl.ds(i * IDXW, IDXW)]],
                    rows_v[slot].at[pl.ds(i * IDXW, IDXW)],
                    sems[slot],
                )
            pltpu.async_copy(wtab_hbm.at[widx_v[slot]],
                             wrows_v[slot], sems[slot])

        def wait_fetch(slot):
            pltpu.make_async_copy(
                ctab_hbm.at[pl.ds(0, ROWS)], rows_v[slot], sems[slot]
            ).wait()
            pltpu.make_async_copy(
                wtab_hbm.at[pl.ds(0, E)], wrows_v[slot], sems[slot]
            ).wait()

        def compute(c, slot):
            base_e = base_e0 + c * E
            lanes = lax.iota(jnp.int32, 16)

            def elem_body(e, _):
                accs = [rows_v[slot][e + g, pl.ds(0, 16)]
                        for g in range(CPAD // 16)]
                for g in range(CPAD // 16):
                    scores_v[e, pl.ds(g * 16, 16)] = accs[g]
                return 0

            lax.fori_loop(0, E, elem_body, 0)
            pltpu.sync_copy(scores_v, out_hbm.at[pl.ds(base_e, E)])

        start_fetch(0, 0)
        start_fetch(1, 1)

        def chunk_body(g, _):
            for b in range(2):
                c = g * 2 + b
                wait_fetch(b)
                compute(c, b)

                @pl.when(c + 2 < NCHUNK)
                def _():
                    start_fetch(c + 2, b)
            return 0

        lax.fori_loop(0, NCHUNK // 2, chunk_body, 0, unroll=False)

    return sc_kernel(words, ctx, w_embedding, c_embedding)


def _tc_loss(scores):
    blk = 2048

    def tc_body(s_ref, o_ref):
        s = s_ref[...]
        j = lax.broadcasted_iota(jnp.int32, s.shape, 1)
        pos = jnp.where(j < P, jax.nn.log_sigmoid(s), 0.0).sum(axis=1) / P
        neg = jnp.where((j >= P) & (j < P + N),
                        jax.nn.log_sigmoid(-s), 0.0).sum(axis=1) / N
        o_ref[...] = -(pos + neg)

    return pl.pallas_call(
        tc_body,
        grid=(B // blk,),
        in_specs=[pl.BlockSpec((blk, CPAD), lambda i: (i, 0))],
        out_specs=pl.BlockSpec((blk,), lambda i: (i,)),
        out_shape=jax.ShapeDtypeStruct((B,), jnp.float32),
    )(scores)


def kernel(words, pos_contexts, neg_contexts, w_embedding, c_embedding):
    pad = jnp.zeros((B, CPAD - P - N), jnp.int32)
    ctx = jnp.concatenate([pos_contexts, neg_contexts, pad],
                          axis=1).reshape(-1)
    scores = _sc_scores(words, ctx, w_embedding, c_embedding)
    return _tc_loss(scores)


# speedup vs baseline: 3.2105x; 1.8135x over previous
import functools

import jax
import jax.numpy as jnp
from jax import lax
from jax.experimental import pallas as pl
from jax.experimental.pallas import tpu as pltpu
from jax.experimental.pallas import tpu_sc as plsc

B = 16384
P = 20
N = 50
DIM = 64
CPAD = 80
E = 8
NW = 32
EPW = B // NW
NCHUNK = EPW // E
ROWS = E * CPAD
IDXW = 128
IDXROWS = ROWS // IDXW


def _sc_scores(words, ctx, w_embedding, c_embedding):
    mesh = plsc.VectorSubcoreMesh(core_axis_name="c", subcore_axis_name="s")

    @functools.partial(
        pl.kernel,
        out_type=jax.ShapeDtypeStruct((B, CPAD), jnp.float32),
        mesh=mesh,
        compiler_params=pltpu.CompilerParams(needs_layout_passes=False,
                                             use_tc_tiling_on_sc=False),
        scratch_types=[
            pltpu.VMEM((ROWS,), jnp.int32),
            pltpu.VMEM((ROWS,), jnp.int32),
            pltpu.VMEM((ROWS, DIM), jnp.float32),
            pltpu.VMEM((ROWS, DIM), jnp.float32),
            pltpu.VMEM((E,), jnp.int32),
            pltpu.VMEM((E,), jnp.int32),
            pltpu.VMEM((E, DIM), jnp.float32),
            pltpu.VMEM((E, DIM), jnp.float32),
            pltpu.VMEM((E, CPAD), jnp.float32),
            pltpu.SemaphoreType.DMA,
            pltpu.SemaphoreType.DMA,
        ],
    )
    def sc_kernel(words_hbm, ctx_hbm, wtab_hbm, ctab_hbm, out_hbm,
                  idx0, idx1, rows0, rows1, widx0, widx1, wrows0, wrows1,
                  scores_v, sem0, sem1):
        idx_v = (idx0, idx1)
        rows_v = (rows0, rows1)
        widx_v = (widx0, widx1)
        wrows_v = (wrows0, wrows1)
        sems = (sem0, sem1)
        wid = lax.axis_index("s") * 2 + lax.axis_index("c")
        base_e0 = wid * EPW

        def start_fetch(c, slot):
            base_e = base_e0 + c * E
            pltpu.sync_copy(ctx_hbm.at[pl.ds(base_e * CPAD, ROWS)],
                            idx_v[slot])
            pltpu.sync_copy(words_hbm.at[pl.ds(base_e, E)], widx_v[slot])
            for i in range(IDXROWS):
                pltpu.async_copy(
                    ctab_hbm.at[idx_v[slot].at[pl.ds(i * IDXW, IDXW)]],
                    rows_v[slot].at[pl.ds(i * IDXW, IDXW)],
                    sems[slot],
                )
            pltpu.async_copy(wtab_hbm.at[widx_v[slot]],
                             wrows_v[slot], sems[slot])

        def wait_fetch(slot):
            pltpu.make_async_copy(
                ctab_hbm.at[pl.ds(0, ROWS)], rows_v[slot], sems[slot]
            ).wait()
            pltpu.make_async_copy(
                wtab_hbm.at[pl.ds(0, E)], wrows_v[slot], sems[slot]
            ).wait()

        def compute(c, slot):
            base_e = base_e0 + c * E
            lanes = lax.iota(jnp.int32, 16)

            def elem_body(e, _):
                accs = [jnp.zeros((16,), jnp.float32)
                        for _ in range(CPAD // 16)]
                row0 = e * CPAD + lanes
                for k in range(DIM // 16):
                    wchunk = wrows_v[slot][e, pl.ds(k * 16, 16)]
                    for i in range(16):
                        d = k * 16 + i
                        wd = wchunk[i]
                        col_idx = jnp.full((16,), d, jnp.int32)
                        for g in range(CPAD // 16):
                            col = plsc.load_gather(
                                rows_v[slot], [row0 + g * 16, col_idx])
                            accs[g] = accs[g] + col * wd
                for g in range(CPAD // 16):
                    scores_v[e, pl.ds(g * 16, 16)] = accs[g]
                return 0

            lax.fori_loop(0, E, elem_body, 0)
            pltpu.sync_copy(scores_v, out_hbm.at[pl.ds(base_e, E)])

        start_fetch(0, 0)
        start_fetch(1, 1)

        def chunk_body(g, _):
            for b in range(2):
                c = g * 2 + b
                wait_fetch(b)
                compute(c, b)

                @pl.when(c + 2 < NCHUNK)
                def _():
                    start_fetch(c + 2, b)
            return 0

        lax.fori_loop(0, NCHUNK // 2, chunk_body, 0, unroll=False)

    return sc_kernel(words, ctx, w_embedding, c_embedding)


def _tc_loss(scores):
    blk = 2048

    def tc_body(s_ref, o_ref):
        s = s_ref[...]
        j = lax.broadcasted_iota(jnp.int32, s.shape, 1)
        pos = jnp.where(j < P, jax.nn.log_sigmoid(s), 0.0).sum(axis=1) / P
        neg = jnp.where((j >= P) & (j < P + N),
                        jax.nn.log_sigmoid(-s), 0.0).sum(axis=1) / N
        o_ref[...] = -(pos + neg)

    return pl.pallas_call(
        tc_body,
        grid=(B // blk,),
        in_specs=[pl.BlockSpec((blk, CPAD), lambda i: (i, 0))],
        out_specs=pl.BlockSpec((blk,), lambda i: (i,)),
        out_shape=jax.ShapeDtypeStruct((B,), jnp.float32),
    )(scores)


def kernel(words, pos_contexts, neg_contexts, w_embedding, c_embedding):
    npad = CPAD - P - N
    pad = (jnp.arange(B, dtype=jnp.int32)[:, None] * npad
           + jnp.arange(npad, dtype=jnp.int32)[None, :]) % (1 << 20)
    ctx = jnp.concatenate([pos_contexts, neg_contexts, pad],
                          axis=1).reshape(-1)
    scores = _sc_scores(words, ctx, w_embedding, c_embedding)
    return _tc_loss(scores)


# speedup vs baseline: 3.2174x; 1.0022x over previous
import functools

import jax
import jax.numpy as jnp
from jax import lax
from jax.experimental import pallas as pl
from jax.experimental.pallas import tpu as pltpu
from jax.experimental.pallas import tpu_sc as plsc

B = 16384
P = 20
N = 50
DIM = 64
CROW = P + N
SCW = 80
E = 8
NW = 32
EPW = B // NW
NCHUNK = EPW // E
ROWS = E * CROW
IDXW = 112
IDXROWS = ROWS // IDXW


def _sc_scores(words, ctx, w_embedding, c_embedding):
    mesh = plsc.VectorSubcoreMesh(core_axis_name="c", subcore_axis_name="s")

    @functools.partial(
        pl.kernel,
        out_type=jax.ShapeDtypeStruct((B, SCW), jnp.float32),
        mesh=mesh,
        compiler_params=pltpu.CompilerParams(needs_layout_passes=False,
                                             use_tc_tiling_on_sc=False),
        scratch_types=[
            pltpu.VMEM((ROWS,), jnp.int32),
            pltpu.VMEM((ROWS,), jnp.int32),
            pltpu.VMEM((ROWS, DIM), jnp.float32),
            pltpu.VMEM((ROWS, DIM), jnp.float32),
            pltpu.VMEM((E,), jnp.int32),
            pltpu.VMEM((E,), jnp.int32),
            pltpu.VMEM((E, DIM), jnp.float32),
            pltpu.VMEM((E, DIM), jnp.float32),
            pltpu.VMEM((E, SCW), jnp.float32),
            pltpu.SemaphoreType.DMA,
            pltpu.SemaphoreType.DMA,
        ],
    )
    def sc_kernel(words_hbm, ctx_hbm, wtab_hbm, ctab_hbm, out_hbm,
                  idx0, idx1, rows0, rows1, widx0, widx1, wrows0, wrows1,
                  scores_v, sem0, sem1):
        idx_v = (idx0, idx1)
        rows_v = (rows0, rows1)
        widx_v = (widx0, widx1)
        wrows_v = (wrows0, wrows1)
        sems = (sem0, sem1)
        wid = lax.axis_index("s") * 2 + lax.axis_index("c")
        base_e0 = wid * EPW

        def start_fetch(c, slot):
            base_e = base_e0 + c * E
            pltpu.sync_copy(ctx_hbm.at[pl.ds(base_e * CROW, ROWS)],
                            idx_v[slot])
            pltpu.sync_copy(words_hbm.at[pl.ds(base_e, E)], widx_v[slot])
            for i in range(IDXROWS):
                pltpu.async_copy(
                    ctab_hbm.at[idx_v[slot].at[pl.ds(i * IDXW, IDXW)]],
                    rows_v[slot].at[pl.ds(i * IDXW, IDXW)],
                    sems[slot],
                )
            pltpu.async_copy(wtab_hbm.at[widx_v[slot]],
                             wrows_v[slot], sems[slot])

        def wait_fetch(slot):
            pltpu.make_async_copy(
                ctab_hbm.at[pl.ds(0, ROWS)], rows_v[slot], sems[slot]
            ).wait()
            pltpu.make_async_copy(
                wtab_hbm.at[pl.ds(0, E)], wrows_v[slot], sems[slot]
            ).wait()

        def compute(c, slot):
            base_e = base_e0 + c * E
            lanes = lax.iota(jnp.int32, 16)

            NG = 5
            def elem_body(e, _):
                accs = [jnp.zeros((16,), jnp.float32) for _ in range(NG)]
                row0 = e * CROW + lanes
                rowidx = [row0 + g * 16 for g in range(NG - 1)]
                rowidx.append(jnp.minimum(row0 + 64, ROWS - 1))
                for k in range(DIM // 16):
                    wchunk = wrows_v[slot][e, pl.ds(k * 16, 16)]
                    for i in range(16):
                        d = k * 16 + i
                        wd = wchunk[i]
                        col_idx = jnp.full((16,), d, jnp.int32)
                        for g in range(NG):
                            col = plsc.load_gather(
                                rows_v[slot], [rowidx[g], col_idx])
                            accs[g] = accs[g] + col * wd
                for g in range(NG):
                    scores_v[e, pl.ds(g * 16, 16)] = accs[g]
                return 0

            lax.fori_loop(0, E, elem_body, 0)
            pltpu.sync_copy(scores_v, out_hbm.at[pl.ds(base_e, E)])

        start_fetch(0, 0)
        start_fetch(1, 1)

        def chunk_body(g, _):
            for b in range(2):
                c = g * 2 + b
                wait_fetch(b)
                compute(c, b)

                @pl.when(c + 2 < NCHUNK)
                def _():
                    start_fetch(c + 2, b)
            return 0

        lax.fori_loop(0, NCHUNK // 2, chunk_body, 0, unroll=False)

    return sc_kernel(words, ctx, w_embedding, c_embedding)


def _tc_loss(scores):
    blk = 2048

    def tc_body(s_ref, o_ref):
        s = s_ref[...]
        j = lax.broadcasted_iota(jnp.int32, s.shape, 1)
        pos = jnp.where(j < P, jax.nn.log_sigmoid(s), 0.0).sum(axis=1) / P
        neg = jnp.where((j >= P) & (j < P + N),
                        jax.nn.log_sigmoid(-s), 0.0).sum(axis=1) / N
        o_ref[...] = -(pos + neg)

    return pl.pallas_call(
        tc_body,
        grid=(B // blk,),
        in_specs=[pl.BlockSpec((blk, SCW), lambda i: (i, 0))],
        out_specs=pl.BlockSpec((blk,), lambda i: (i,)),
        out_shape=jax.ShapeDtypeStruct((B,), jnp.float32),
    )(scores)


def kernel(words, pos_contexts, neg_contexts, w_embedding, c_embedding):
    ctx = jnp.concatenate([pos_contexts, neg_contexts], axis=1).reshape(-1)
    scores = _sc_scores(words, ctx, w_embedding, c_embedding)
    return _tc_loss(scores)


# speedup vs baseline: 3.3072x; 1.0279x over previous
import functools

import jax
import jax.numpy as jnp
from jax import lax
from jax.experimental import pallas as pl
from jax.experimental.pallas import tpu as pltpu
from jax.experimental.pallas import tpu_sc as plsc

B = 16384
P = 20
N = 50
DIM = 64
CROW = P + N
SCW = 80
E = 8
NW = 32
EPW = B // NW
NCHUNK = EPW // E
ROWS = E * CROW
IDXW = 112
IDXROWS = ROWS // IDXW
NG = 5


def _sc_scores(words, ctx, w_embedding, c_embedding):
    mesh = plsc.VectorSubcoreMesh(core_axis_name="c", subcore_axis_name="s")

    @functools.partial(
        pl.kernel,
        out_type=jax.ShapeDtypeStruct((B, SCW), jnp.float32),
        mesh=mesh,
        compiler_params=pltpu.CompilerParams(needs_layout_passes=False,
                                             use_tc_tiling_on_sc=False),
        scratch_types=[
            pltpu.VMEM((EPW * CROW,), jnp.int32),
            pltpu.VMEM((EPW,), jnp.int32),
            pltpu.VMEM((ROWS, DIM), jnp.float32),
            pltpu.VMEM((ROWS, DIM), jnp.float32),
            pltpu.VMEM((E, DIM), jnp.float32),
            pltpu.VMEM((E, DIM), jnp.float32),
            pltpu.VMEM((E, SCW), jnp.float32),
            pltpu.VMEM((E, SCW), jnp.float32),
            pltpu.SemaphoreType.DMA,
            pltpu.SemaphoreType.DMA,
            pltpu.SemaphoreType.DMA,
            pltpu.SemaphoreType.DMA,
        ],
    )
    def sc_kernel(words_hbm, ctx_hbm, wtab_hbm, ctab_hbm, out_hbm,
                  idx_v, widx_v, rows0, rows1, wrows0, wrows1,
                  scores0, scores1, semf0, semf1, semo0, semo1):
        rows_v = (rows0, rows1)
        wrows_v = (wrows0, wrows1)
        scores_v = (scores0, scores1)
        semf = (semf0, semf1)
        semo = (semo0, semo1)
        wid = lax.axis_index("s") * 2 + lax.axis_index("c")
        base_e0 = wid * EPW

        def start_fetch(c, slot):
            off = c * ROWS
            for i in range(IDXROWS):
                pltpu.async_copy(
                    ctab_hbm.at[idx_v.at[pl.ds(off + i * IDXW, IDXW)]],
                    rows_v[slot].at[pl.ds(i * IDXW, IDXW)],
                    semf[slot],
                )
            pltpu.async_copy(wtab_hbm.at[widx_v.at[pl.ds(c * E, E)]],
                             wrows_v[slot], semf[slot])

        def wait_fetch(slot):
            pltpu.make_async_copy(
                ctab_hbm.at[pl.ds(0, ROWS)], rows_v[slot], semf[slot]
            ).wait()
            pltpu.make_async_copy(
                wtab_hbm.at[pl.ds(0, E)], wrows_v[slot], semf[slot]
            ).wait()

        def drain_out(slot):
            pltpu.make_async_copy(
                scores_v[slot], out_hbm.at[pl.ds(0, E)], semo[slot]
            ).wait()

        def compute(slot):
            lanes = lax.iota(jnp.int32, 16)

            def elem_body(e, _):
                accs = [jnp.zeros((16,), jnp.float32) for _ in range(NG)]
                row0 = e * CROW + lanes
                rowidx = [row0 + g * 16 for g in range(NG - 1)]
                rowidx.append(jnp.minimum(row0 + 64, ROWS - 1))
                for k in range(DIM // 16):
                    wchunk = wrows_v[slot][e, pl.ds(k * 16, 16)]
                    for i in range(16):
                        d = k * 16 + i
                        wd = wchunk[i]
                        col_idx = jnp.full((16,), d, jnp.int32)
                        for g in range(NG):
                            col = plsc.load_gather(
                                rows_v[slot], [rowidx[g], col_idx])
                            accs[g] = accs[g] + col * wd
                for g in range(NG):
                    scores_v[slot][e, pl.ds(g * 16, 16)] = accs[g]
                return 0

            lax.fori_loop(0, E, elem_body, 0)

        pltpu.sync_copy(ctx_hbm.at[pl.ds(base_e0 * CROW, EPW * CROW)], idx_v)
        pltpu.sync_copy(words_hbm.at[pl.ds(base_e0, EPW)], widx_v)
        start_fetch(0, 0)
        start_fetch(1, 1)

        def chunk_body(g, _):
            for b in range(2):
                c = g * 2 + b
                wait_fetch(b)

                @pl.when(c >= 2)
                def _():
                    drain_out(b)

                compute(b)
                pltpu.async_copy(
                    scores_v[b],
                    out_hbm.at[pl.ds(base_e0 + c * E, E)],
                    semo[b],
                )

                @pl.when(c + 2 < NCHUNK)
                def _():
                    start_fetch(c + 2, b)
            return 0

        lax.fori_loop(0, NCHUNK // 2, chunk_body, 0, unroll=False)
        drain_out(0)
        drain_out(1)

    return sc_kernel(words, ctx, w_embedding, c_embedding)


def _tc_loss(scores):
    blk = 2048

    def tc_body(s_ref, o_ref):
        s = s_ref[...]
        j = lax.broadcasted_iota(jnp.int32, s.shape, 1)
        pos = jnp.where(j < P, jax.nn.log_sigmoid(s), 0.0).sum(axis=1) / P
        neg = jnp.where((j >= P) & (j < P + N),
                        jax.nn.log_sigmoid(-s), 0.0).sum(axis=1) / N
        o_ref[...] = -(pos + neg)

    return pl.pallas_call(
        tc_body,
        grid=(B // blk,),
        in_specs=[pl.BlockSpec((blk, SCW), lambda i: (i, 0))],
        out_specs=pl.BlockSpec((blk,), lambda i: (i,)),
        out_shape=jax.ShapeDtypeStruct((B,), jnp.float32),
    )(scores)


def kernel(words, pos_contexts, neg_contexts, w_embedding, c_embedding):
    ctx = jnp.concatenate([pos_contexts, neg_contexts], axis=1).reshape(-1)
    scores = _sc_scores(words, ctx, w_embedding, c_embedding)
    return _tc_loss(scores)
